# Initial kernel scaffold; baseline (speedup 1.0000x reference)
import jax
import jax.numpy as jnp
from jax.experimental import pallas as pl


def kernel(efeat, nfeat, edge_index, edge_w1, edge_b1, edge_w2, edge_b2, edge_ln_s, edge_ln_b, node_w1, node_b1, node_w2, node_b2, node_ln_s, node_ln_b):
    raise NotImplementedError("write your pallas kernel here")


# speedup vs baseline: 3.2521x; 3.2521x over previous
import functools

import jax
import jax.numpy as jnp
from jax import lax
from jax.experimental import pallas as pl
from jax.experimental.pallas import tpu as pltpu
from jax.experimental.pallas import tpu_sc as plsc

L = 2
N = 10000
E = 160000
D = 128

NC = 2
NS = 16
NW = NC * NS
CH = 128
NCHUNK = E // CH
NPAD = 10240
NPS = NPAD // NS


def _silu(x):
  return x * jax.nn.sigmoid(x)


def _ln_res(base, o, s, b):
  m = jnp.mean(o, axis=-1, keepdims=True)
  v = jnp.mean((o - m) ** 2, axis=-1, keepdims=True)
  return base + (o - m) * lax.rsqrt(v + 1e-5) * s + b


def _pre_body(n_ref, w1s_ref, w1d_ref, b1_ref, as_ref, ad_ref):
  n = n_ref[...]
  as_ref[...] = jnp.dot(n, w1s_ref[...], preferred_element_type=jnp.float32) + b1_ref[...]
  ad_ref[...] = jnp.dot(n, w1d_ref[...], preferred_element_type=jnp.float32)


def _edge_body(e_ref, g_ref, w1e_ref, w2_ref, b2_ref, s_ref, b_ref, out_ref):
  e = e_ref[...]
  pre = jnp.dot(e, w1e_ref[...], preferred_element_type=jnp.float32) + g_ref[...]
  o = jnp.dot(_silu(pre), w2_ref[...], preferred_element_type=jnp.float32) + b2_ref[...]
  out_ref[...] = _ln_res(e, o, s_ref[...], b_ref[...])


def _node_body(p0_ref, p1_ref, n_ref, w1a_ref, w1n_ref, b1_ref, w2_ref,
               b2_ref, s_ref, b_ref, out_ref):
  n = n_ref[...]
  agg = p0_ref[...] + p1_ref[...]
  pre = (jnp.dot(agg, w1a_ref[...], preferred_element_type=jnp.float32)
         + jnp.dot(n, w1n_ref[...], preferred_element_type=jnp.float32)
         + b1_ref[...])
  o = jnp.dot(_silu(pre), w2_ref[...], preferred_element_type=jnp.float32) + b2_ref[...]
  out_ref[...] = _ln_res(n, o, s_ref[...], b_ref[...])


def _row_spec(bn):
  return pl.BlockSpec((bn, D), lambda i: (i, 0))


_W = pl.BlockSpec((D, D), lambda i: (0, 0))
_V = pl.BlockSpec((1, D), lambda i: (0, 0))


def _pre_call(n, w1s, w1d, b1):
  bn = 2000
  return pl.pallas_call(
      _pre_body,
      grid=(N // bn,),
      in_specs=[_row_spec(bn), _W, _W, _V],
      out_specs=[_row_spec(bn), _row_spec(bn)],
      out_shape=[jax.ShapeDtypeStruct((N, D), jnp.float32)] * 2,
  )(n, w1s, w1d, b1)


def _edge_call(e, g, w1e, w2, b2, ln_s, ln_b):
  be = 2000
  return pl.pallas_call(
      _edge_body,
      grid=(E // be,),
      in_specs=[_row_spec(be), _row_spec(be), _W, _W, _V, _V, _V],
      out_specs=_row_spec(be),
      out_shape=jax.ShapeDtypeStruct((E, D), jnp.float32),
  )(e, g, w1e, w2, b2, ln_s, ln_b)


def _node_call(p0, p1, n, w1a, w1n, b1, w2, b2, ln_s, ln_b):
  bn = 2000
  return pl.pallas_call(
      _node_body,
      grid=(N // bn,),
      in_specs=[_row_spec(bn), _row_spec(bn), _row_spec(bn), _W, _W, _V, _W,
                _V, _V, _V],
      out_specs=_row_spec(bn),
      out_shape=jax.ShapeDtypeStruct((N, D), jnp.float32),
  )(p0, p1, n, w1a, w1n, b1, w2, b2, ln_s, ln_b)


_SC_MESH = plsc.VectorSubcoreMesh(
    core_axis_name="c", subcore_axis_name="s", num_cores=NC, num_subcores=NS)


def _gather_body(as_hbm, ad_hbm, src_hbm, dst_hbm, out_hbm,
                 sidx, didx, ra, rb, sa, sb):
  wid = lax.axis_index("s") * NC + lax.axis_index("c")
  nchunks = jnp.where(wid < NCHUNK - (NCHUNK // NW) * NW, NCHUNK // NW + 1,
                      NCHUNK // NW)

  def chunk(j, carry):
    cid = wid + j * NW
    base = pl.multiple_of(cid * CH, CH)
    pltpu.sync_copy(src_hbm.at[pl.ds(base, CH)], sidx)
    pltpu.sync_copy(dst_hbm.at[pl.ds(base, CH)], didx)
    ca = pltpu.async_copy(as_hbm.at[sidx], ra, sa)
    cb = pltpu.async_copy(ad_hbm.at[didx], rb, sb)
    ca.wait()
    cb.wait()

    def addrow(r, c2):
      for c in range(D // 16):
        ra[r, pl.ds(c * 16, 16)] = (ra[r, pl.ds(c * 16, 16)]
                                    + rb[r, pl.ds(c * 16, 16)])
      return c2

    lax.fori_loop(0, CH, addrow, 0)
    pltpu.sync_copy(ra, out_hbm.at[pl.ds(base, CH)])
    return carry

  lax.fori_loop(0, nchunks, chunk, 0)


_gather_call = pl.kernel(
    _gather_body,
    out_type=jax.ShapeDtypeStruct((E, D), jnp.float32),
    mesh=_SC_MESH,
    scratch_types=[
        pltpu.VMEM((CH,), jnp.int32),
        pltpu.VMEM((CH,), jnp.int32),
        pltpu.VMEM((CH, D), jnp.float32),
        pltpu.VMEM((CH, D), jnp.float32),
        pltpu.SemaphoreType.DMA,
        pltpu.SemaphoreType.DMA,
    ],
)


def _scatter_body(e_hbm, dst_hbm, zero_hbm, out_hbm, didx, rows, accum):
  c = lax.axis_index("c")
  s = lax.axis_index("s")
  wid = s * NC + c
  pltpu.sync_copy(zero_hbm.at[pl.ds(s * NPS, NPS)],
                  accum.at[pl.ds(s * NPS, NPS)])
  plsc.subcore_barrier()

  nchunks = jnp.where(wid < NCHUNK - (NCHUNK // NW) * NW, NCHUNK // NW + 1,
                      NCHUNK // NW)

  def chunk(j, carry):
    cid = wid + j * NW
    base = pl.multiple_of(cid * CH, CH)
    pltpu.sync_copy(dst_hbm.at[pl.ds(base, CH)], didx)
    pltpu.sync_copy(e_hbm.at[pl.ds(base, CH)], rows)
    pltpu.sync_copy(rows, accum.at[didx], add=True)
    return carry

  lax.fori_loop(0, nchunks, chunk, 0)
  plsc.subcore_barrier()
  pltpu.sync_copy(accum.at[pl.ds(s * NPS, NPS)],
                  out_hbm.at[c, pl.ds(s * NPS, NPS)])


_scatter_call = pl.kernel(
    _scatter_body,
    out_type=jax.ShapeDtypeStruct((NC, NPAD, D), jnp.float32),
    mesh=_SC_MESH,
    scratch_types=[
        pltpu.VMEM((CH,), jnp.int32),
        pltpu.VMEM((CH, D), jnp.float32),
        pltpu.VMEM_SHARED((NPAD, D), jnp.float32),
    ],
)


def kernel(efeat, nfeat, edge_index, edge_w1, edge_b1, edge_w2, edge_b2,
           edge_ln_s, edge_ln_b, node_w1, node_b1, node_w2, node_b2,
           node_ln_s, node_ln_b):
  src = edge_index[0]
  dst = edge_index[1]
  zero = jnp.zeros((NPAD, D), jnp.float32)
  e, n = efeat, nfeat
  for i in range(L):
    w1e = edge_w1[i, :D]
    w1s = edge_w1[i, D:2 * D]
    w1d = edge_w1[i, 2 * D:]
    a_s, a_d = _pre_call(n, w1s, w1d, edge_b1[i].reshape(1, D))
    g = _gather_call(a_s, a_d, src, dst)
    e = _edge_call(e, g, w1e, edge_w2[i], edge_b2[i].reshape(1, D),
                   edge_ln_s[i].reshape(1, D), edge_ln_b[i].reshape(1, D))
    parts = _scatter_call(e, dst, zero)
    n = _node_call(parts[0], parts[1], n, node_w1[i, :D], node_w1[i, D:],
                   node_b1[i].reshape(1, D), node_w2[i],
                   node_b2[i].reshape(1, D), node_ln_s[i].reshape(1, D),
                   node_ln_b[i].reshape(1, D))
  return (e, n)


# speedup vs baseline: 4.7847x; 1.4712x over previous
import functools

import jax
import jax.numpy as jnp
from jax import lax
from jax.experimental import pallas as pl
from jax.experimental.pallas import tpu as pltpu
from jax.experimental.pallas import tpu_sc as plsc

L = 2
N = 10000
E = 160000
D = 128

NC = 2
NS = 16
NW = NC * NS
CH = 128
NCHUNK = E // CH
NPAD = 10240
NPS = NPAD // NS


def _silu(x):
  return x * jax.nn.sigmoid(x)


def _ln_res(base, o, s, b):
  m = jnp.mean(o, axis=-1, keepdims=True)
  v = jnp.mean((o - m) ** 2, axis=-1, keepdims=True)
  return base + (o - m) * lax.rsqrt(v + 1e-5) * s + b


def _pre_body(n_ref, w1s_ref, w1d_ref, b1_ref, as_ref, ad_ref):
  n = n_ref[...]
  as_ref[...] = jnp.dot(n, w1s_ref[...], preferred_element_type=jnp.float32) + b1_ref[...]
  ad_ref[...] = jnp.dot(n, w1d_ref[...], preferred_element_type=jnp.float32)


def _edge_body(e_ref, g_ref, w1e_ref, w2_ref, b2_ref, s_ref, b_ref, out_ref):
  e = e_ref[...]
  pre = jnp.dot(e, w1e_ref[...], preferred_element_type=jnp.float32) + g_ref[...]
  o = jnp.dot(_silu(pre), w2_ref[...], preferred_element_type=jnp.float32) + b2_ref[...]
  out_ref[...] = _ln_res(e, o, s_ref[...], b_ref[...])


def _node_body(p0_ref, p1_ref, n_ref, w1a_ref, w1n_ref, b1_ref, w2_ref,
               b2_ref, s_ref, b_ref, out_ref):
  n = n_ref[...]
  agg = p0_ref[...] + p1_ref[...]
  pre = (jnp.dot(agg, w1a_ref[...], preferred_element_type=jnp.float32)
         + jnp.dot(n, w1n_ref[...], preferred_element_type=jnp.float32)
         + b1_ref[...])
  o = jnp.dot(_silu(pre), w2_ref[...], preferred_element_type=jnp.float32) + b2_ref[...]
  out_ref[...] = _ln_res(n, o, s_ref[...], b_ref[...])


def _row_spec(bn):
  return pl.BlockSpec((bn, D), lambda i: (i, 0))


_W = pl.BlockSpec((D, D), lambda i: (0, 0))
_V = pl.BlockSpec((1, D), lambda i: (0, 0))


def _pre_call(n, w1s, w1d, b1):
  bn = 2000
  return pl.pallas_call(
      _pre_body,
      grid=(N // bn,),
      in_specs=[_row_spec(bn), _W, _W, _V],
      out_specs=[_row_spec(bn), _row_spec(bn)],
      out_shape=[jax.ShapeDtypeStruct((N, D), jnp.float32)] * 2,
  )(n, w1s, w1d, b1)


def _edge_call(e, g, w1e, w2, b2, ln_s, ln_b):
  be = 2000
  return pl.pallas_call(
      _edge_body,
      grid=(E // be,),
      in_specs=[_row_spec(be), _row_spec(be), _W, _W, _V, _V, _V],
      out_specs=_row_spec(be),
      out_shape=jax.ShapeDtypeStruct((E, D), jnp.float32),
  )(e, g, w1e, w2, b2, ln_s, ln_b)


def _node_call(p0, p1, n, w1a, w1n, b1, w2, b2, ln_s, ln_b):
  bn = 2000
  return pl.pallas_call(
      _node_body,
      grid=(N // bn,),
      in_specs=[_row_spec(bn), _row_spec(bn), _row_spec(bn), _W, _W, _V, _W,
                _V, _V, _V],
      out_specs=_row_spec(bn),
      out_shape=jax.ShapeDtypeStruct((N, D), jnp.float32),
  )(p0, p1, n, w1a, w1n, b1, w2, b2, ln_s, ln_b)


_SC_MESH = plsc.VectorSubcoreMesh(
    core_axis_name="c", subcore_axis_name="s", num_cores=NC, num_subcores=NS)


EPT = E // NW
NCT = EPT // CH
TAIL = EPT - NCT * CH


def _gather_body(as_hbm, ad_hbm, src_hbm, dst_hbm, out_hbm,
                 isrc, idst, ra, rb, wo, sg, sw, st):
  wid = lax.axis_index("s") * NC + lax.axis_index("c")
  tb = pl.multiple_of(wid * EPT, 8)
  pltpu.sync_copy(src_hbm.at[pl.ds(tb, EPT)], isrc)
  pltpu.sync_copy(dst_hbm.at[pl.ds(tb, EPT)], idst)

  def fire(j, b):
    pltpu.async_copy(as_hbm.at[isrc.at[pl.ds(j * CH, CH)]], ra.at[b],
                     sg.at[b])
    pltpu.async_copy(ad_hbm.at[idst.at[pl.ds(j * CH, CH)]], rb.at[b],
                     sg.at[b])

  def consume(j, b):
    base = pl.multiple_of(tb + j * CH, 8)
    pltpu.make_async_copy(as_hbm.at[isrc.at[pl.ds(j * CH, CH)]], ra.at[b],
                          sg.at[b]).wait()
    pltpu.make_async_copy(ad_hbm.at[idst.at[pl.ds(j * CH, CH)]], rb.at[b],
                          sg.at[b]).wait()

    @plsc.parallel_loop(0, CH, unroll=4)
    def _add(r):
      for c in range(D // 16):
        wo[b, r, pl.ds(c * 16, 16)] = (ra[b, r, pl.ds(c * 16, 16)]
                                       + rb[b, r, pl.ds(c * 16, 16)])

    pltpu.async_copy(wo.at[b], out_hbm.at[pl.ds(base, CH)], sw.at[b])

  def drain_w(b):
    pltpu.make_async_copy(wo.at[b], out_hbm.at[pl.ds(tb, CH)],
                          sw.at[b]).wait()

  fire(0, 0)

  def body(j, carry):
    nb = (j + 1) % 2
    b = j % 2

    @pl.when(j + 1 < NCT)
    def _pref():
      @pl.when(j >= 1)
      def _():
        drain_w(nb)
      fire(j + 1, nb)

    consume(j, b)
    return carry

  lax.fori_loop(0, NCT, body, 0)
  jt = NCT * CH
  baset = pl.multiple_of(tb + jt, 8)
  pltpu.async_copy(as_hbm.at[isrc.at[pl.ds(jt, TAIL)]],
                   ra.at[0, pl.ds(0, TAIL)], st)
  pltpu.async_copy(ad_hbm.at[idst.at[pl.ds(jt, TAIL)]],
                   rb.at[0, pl.ds(0, TAIL)], st)
  drain_w(0)
  drain_w(1)
  pltpu.make_async_copy(as_hbm.at[isrc.at[pl.ds(jt, TAIL)]],
                        ra.at[0, pl.ds(0, TAIL)], st).wait()
  pltpu.make_async_copy(ad_hbm.at[idst.at[pl.ds(jt, TAIL)]],
                        rb.at[0, pl.ds(0, TAIL)], st).wait()

  @plsc.parallel_loop(0, TAIL)
  def _addt(r):
    for c in range(D // 16):
      wo[0, r, pl.ds(c * 16, 16)] = (ra[0, r, pl.ds(c * 16, 16)]
                                     + rb[0, r, pl.ds(c * 16, 16)])

  pltpu.sync_copy(wo.at[0, pl.ds(0, TAIL)], out_hbm.at[pl.ds(baset, TAIL)])


_gather_call = pl.kernel(
    _gather_body,
    out_type=jax.ShapeDtypeStruct((E, D), jnp.float32),
    mesh=_SC_MESH,
    scratch_types=[
        pltpu.VMEM((EPT,), jnp.int32),
        pltpu.VMEM((EPT,), jnp.int32),
        pltpu.VMEM((2, CH, D), jnp.float32),
        pltpu.VMEM((2, CH, D), jnp.float32),
        pltpu.VMEM((2, CH, D), jnp.float32),
        pltpu.SemaphoreType.DMA((2,)),
        pltpu.SemaphoreType.DMA((2,)),
        pltpu.SemaphoreType.DMA,
    ],
)


def _scatter_body(e_hbm, dst_hbm, zero_hbm, out_hbm,
                  didx, tidx, rows, accum, si, ss, st):
  c = lax.axis_index("c")
  s = lax.axis_index("s")
  wid = s * NC + c
  tb = pl.multiple_of(wid * EPT, 8)
  pltpu.sync_copy(zero_hbm.at[pl.ds(s * NPS, NPS)],
                  accum.at[pl.ds(s * NPS, NPS)])
  plsc.subcore_barrier()

  def fire_in(j, b):
    base = pl.multiple_of(tb + j * CH, 8)
    pltpu.async_copy(dst_hbm.at[pl.ds(base, CH)], didx.at[b], si.at[b])
    pltpu.async_copy(e_hbm.at[pl.ds(base, CH)], rows.at[b], si.at[b])

  def wait_in(j, b):
    base = pl.multiple_of(tb + j * CH, 8)
    pltpu.make_async_copy(dst_hbm.at[pl.ds(base, CH)], didx.at[b],
                          si.at[b]).wait()
    pltpu.make_async_copy(e_hbm.at[pl.ds(base, CH)], rows.at[b],
                          si.at[b]).wait()

  def drain_sc(b):
    pltpu.make_async_copy(rows.at[b], accum.at[didx.at[b]], ss.at[b]).wait()

  fire_in(0, 0)

  def body(j, carry):
    nb = (j + 1) % 2
    b = j % 2

    @pl.when(j + 1 < NCT)
    def _pref():
      @pl.when(j >= 1)
      def _():
        drain_sc(nb)
      fire_in(j + 1, nb)

    wait_in(j, b)
    pltpu.async_copy(rows.at[b], accum.at[didx.at[b]], ss.at[b], add=True)
    return carry

  lax.fori_loop(0, NCT, body, 0)
  baset = pl.multiple_of(tb + NCT * CH, 8)
  drain_sc(0)
  drain_sc(1)
  pltpu.sync_copy(dst_hbm.at[pl.ds(baset, TAIL)], tidx)
  pltpu.sync_copy(e_hbm.at[pl.ds(baset, TAIL)], rows.at[0, pl.ds(0, TAIL)])
  pltpu.sync_copy(rows.at[0, pl.ds(0, TAIL)], accum.at[tidx], add=True)
  plsc.subcore_barrier()
  pltpu.sync_copy(accum.at[pl.ds(s * NPS, NPS)],
                  out_hbm.at[c, pl.ds(s * NPS, NPS)])


_scatter_call = pl.kernel(
    _scatter_body,
    out_type=jax.ShapeDtypeStruct((NC, NPAD, D), jnp.float32),
    mesh=_SC_MESH,
    scratch_types=[
        pltpu.VMEM((2, CH), jnp.int32),
        pltpu.VMEM((TAIL,), jnp.int32),
        pltpu.VMEM((2, CH, D), jnp.float32),
        pltpu.VMEM_SHARED((NPAD, D), jnp.float32),
        pltpu.SemaphoreType.DMA((2,)),
        pltpu.SemaphoreType.DMA((2,)),
        pltpu.SemaphoreType.DMA,
    ],
)


def kernel(efeat, nfeat, edge_index, edge_w1, edge_b1, edge_w2, edge_b2,
           edge_ln_s, edge_ln_b, node_w1, node_b1, node_w2, node_b2,
           node_ln_s, node_ln_b):
  src = edge_index[0]
  dst = edge_index[1]
  zero = jnp.zeros((NPAD, D), jnp.float32)
  e, n = efeat, nfeat
  for i in range(L):
    w1e = edge_w1[i, :D]
    w1s = edge_w1[i, D:2 * D]
    w1d = edge_w1[i, 2 * D:]
    a_s, a_d = _pre_call(n, w1s, w1d, edge_b1[i].reshape(1, D))
    g = _gather_call(a_s, a_d, src, dst)
    e = _edge_call(e, g, w1e, edge_w2[i], edge_b2[i].reshape(1, D),
                   edge_ln_s[i].reshape(1, D), edge_ln_b[i].reshape(1, D))
    parts = _scatter_call(e, dst, zero)
    n = _node_call(parts[0], parts[1], n, node_w1[i, :D], node_w1[i, D:],
                   node_b1[i].reshape(1, D), node_w2[i],
                   node_b2[i].reshape(1, D), node_ln_s[i].reshape(1, D),
                   node_ln_b[i].reshape(1, D))
  return (e, n)
